# Initial kernel scaffold; baseline (speedup 1.0000x reference)
import jax
import jax.numpy as jnp
from jax.experimental import pallas as pl


def kernel(embs, merge_inputs, position_table):
    raise NotImplementedError("write your pallas kernel here")


# speedup vs baseline: 4.5511x; 4.5511x over previous
import jax
import jax.numpy as jnp
from jax import lax
from jax.experimental import pallas as pl

_B, _L, _D, _V = 4096, 200, 64, 512
_N = _B * _L
_RB = 6400
_GRID = _N // _RB


def _tc_body(idx_ref, embs_ref, table_ref, out_ref):
    idxv = idx_ref[0, 0, :]
    iota = lax.broadcasted_iota(jnp.int32, (_V, _RB), 0)
    onehot = jnp.where(idxv[None, :] == iota,
                       jnp.float32(1), jnp.float32(0)).astype(jnp.bfloat16)
    table = table_ref[...].astype(jnp.bfloat16)
    pe = lax.dot_general(onehot, table, (((0,), (0,)), ((), ())),
                         preferred_element_type=jnp.float32)
    out_ref[...] = embs_ref[...] + pe


def kernel(embs, merge_inputs, position_table):
    idx = merge_inputs.astype(jnp.int32).reshape(_GRID, 1, _RB)
    embs2 = embs.reshape(_N, _D)
    out = pl.pallas_call(
        _tc_body,
        grid=(_GRID,),
        in_specs=[
            pl.BlockSpec((1, 1, _RB), lambda i: (i, 0, 0)),
            pl.BlockSpec((_RB, _D), lambda i: (i, 0)),
            pl.BlockSpec((_V, _D), lambda i: (0, 0)),
        ],
        out_specs=pl.BlockSpec((_RB, _D), lambda i: (i, 0)),
        out_shape=jax.ShapeDtypeStruct((_N, _D), jnp.float32),
    )(idx, embs2, position_table)
    return out.reshape(_B, _L, _D)
